# Initial kernel scaffold; baseline (speedup 1.0000x reference)
import jax
import jax.numpy as jnp
from jax.experimental import pallas as pl


def kernel(g_feats, edge_index, W_in, b_in, Ws1, bs1, Wd1, bd1, a1, Ws2, bs2, Wd2, bd2, a2, Wh1, bh1, Wh2, bh2):
    raise NotImplementedError("write your pallas kernel here")


# speedup vs baseline: 2.1887x; 2.1887x over previous
import functools

import jax
import jax.numpy as jnp
from jax import lax
from jax.experimental import pallas as pl
from jax.experimental.pallas import tpu as pltpu
from jax.experimental.pallas import tpu_sc as plsc

N = 10000
E = 320000
DIN = 128
DH = 64
H = 8
NCLS = 10
F = H * DH

NW = 32
EPW = E // NW
K = 80
NCH = EPW // K
NPS = 640
NPAD = 16 * NPS

f32 = jnp.float32
BN = 1000
BE = 2000


def _mm_in_body(x_ref, win_ref, bin_ref, ws_ref, bs_ref, wd_ref, bd_ref,
                fs_ref, fd_ref):
    h0 = jnp.dot(x_ref[...], win_ref[...], preferred_element_type=f32)
    h0 = h0 + bin_ref[...]
    fs_ref[...] = jnp.dot(h0, ws_ref[...], preferred_element_type=f32) + bs_ref[...]
    fd_ref[...] = jnp.dot(h0, wd_ref[...], preferred_element_type=f32) + bd_ref[...]


def _mm_in(x, Win, bin_, Ws, bs, Wd, bd):
    return pl.pallas_call(
        _mm_in_body,
        grid=(N // BN,),
        in_specs=[
            pl.BlockSpec((BN, DIN), lambda i: (i, 0)),
            pl.BlockSpec((DIN, DH), lambda i: (0, 0)),
            pl.BlockSpec((1, DH), lambda i: (0, 0)),
            pl.BlockSpec((DH, F), lambda i: (0, 0)),
            pl.BlockSpec((1, F), lambda i: (0, 0)),
            pl.BlockSpec((DH, F), lambda i: (0, 0)),
            pl.BlockSpec((1, F), lambda i: (0, 0)),
        ],
        out_specs=[pl.BlockSpec((BN, F), lambda i: (i, 0))] * 2,
        out_shape=[jax.ShapeDtypeStruct((N, F), f32)] * 2,
    )(x, Win, bin_.reshape(1, DH), Ws, bs.reshape(1, F), Wd, bd.reshape(1, F))


def _edge_body(gs_ref, gd_ref, a_ref, msg_ref, ex_ref):
    x = gs_ref[...] + gd_ref[...]
    t = jnp.maximum(x, 0.2 * x)
    logits = jnp.dot(t, a_ref[...], preferred_element_type=f32)
    col = lax.broadcasted_iota(jnp.int32, (1, 16), 1)
    ex = jnp.where(col < H, jnp.exp(logits), 0.0)
    ex_ref[...] = ex
    for h in range(H):
        msg_ref[h] = gs_ref[:, h * DH:(h + 1) * DH] * ex[:, h:h + 1]


def _edge_pass(gs, gd, a16):
    return pl.pallas_call(
        _edge_body,
        grid=(E // BE,),
        in_specs=[
            pl.BlockSpec((BE, F), lambda i: (i, 0)),
            pl.BlockSpec((BE, F), lambda i: (i, 0)),
            pl.BlockSpec((F, 16), lambda i: (0, 0)),
        ],
        out_specs=[
            pl.BlockSpec((H, BE, DH), lambda i: (0, i, 0)),
            pl.BlockSpec((BE, 16), lambda i: (i, 0)),
        ],
        out_shape=[
            jax.ShapeDtypeStruct((H, E, DH), f32),
            jax.ShapeDtypeStruct((E, 16), f32),
        ],
    )(gs, gd, a16)


def _heads_merge(acc_ref, den_ref):
    d = den_ref[0] + den_ref[1]
    s = jnp.zeros((acc_ref.shape[2], DH), f32)
    for h in range(H):
        num = acc_ref[h, 0] + acc_ref[h, 1]
        dh = d[:, h:h + 1]
        s = s + jnp.where(dh > 0, num / dh, 0.0)
    return s * (1.0 / H)


def _node_mid_body(acc_ref, den_ref, ws_ref, bs_ref, wd_ref, bd_ref,
                   fs_ref, fd_ref):
    h1 = jnp.maximum(_heads_merge(acc_ref, den_ref), 0.0)
    fs_ref[...] = jnp.dot(h1, ws_ref[...], preferred_element_type=f32) + bs_ref[...]
    fd_ref[...] = jnp.dot(h1, wd_ref[...], preferred_element_type=f32) + bd_ref[...]


def _node_mid(acc, den, Ws, bs, Wd, bd):
    return pl.pallas_call(
        _node_mid_body,
        grid=(N // BN,),
        in_specs=[
            pl.BlockSpec((H, 2, BN, DH), lambda i: (0, 0, i, 0)),
            pl.BlockSpec((2, BN, 16), lambda i: (0, i, 0)),
            pl.BlockSpec((DH, F), lambda i: (0, 0)),
            pl.BlockSpec((1, F), lambda i: (0, 0)),
            pl.BlockSpec((DH, F), lambda i: (0, 0)),
            pl.BlockSpec((1, F), lambda i: (0, 0)),
        ],
        out_specs=[pl.BlockSpec((BN, F), lambda i: (i, 0))] * 2,
        out_shape=[jax.ShapeDtypeStruct((N, F), f32)] * 2,
    )(acc, den, Ws, bs.reshape(1, F), Wd, bd.reshape(1, F))


def _reduce_body(acc_ref, den_ref, o_ref):
    h2 = _heads_merge(acc_ref, den_ref)
    part = jnp.sum(h2, axis=0, keepdims=True)

    @pl.when(pl.program_id(0) == 0)
    def _():
        o_ref[...] = jnp.zeros_like(o_ref)

    o_ref[...] += part


def _reduce_nodes(acc, den):
    return pl.pallas_call(
        _reduce_body,
        grid=(N // BN,),
        in_specs=[
            pl.BlockSpec((H, 2, BN, DH), lambda i: (0, 0, i, 0)),
            pl.BlockSpec((2, BN, 16), lambda i: (0, i, 0)),
        ],
        out_specs=pl.BlockSpec((1, DH), lambda i: (0, 0)),
        out_shape=jax.ShapeDtypeStruct((1, DH), f32),
    )(acc, den)


def _head_body(hs_ref, w1_ref, b1_ref, w2_ref, b2_ref, o_ref):
    g = jnp.broadcast_to(hs_ref[...] * (1.0 / N), (8, DH))
    a = jnp.maximum(jnp.dot(g, w1_ref[...], preferred_element_type=f32)
                    + b1_ref[...], 0.0)
    z = jnp.dot(a, w2_ref[...], preferred_element_type=f32) + b2_ref[...]
    z = z - jnp.max(z, axis=-1, keepdims=True)
    ez = jnp.exp(z)
    sm = ez / jnp.sum(ez, axis=-1, keepdims=True)
    o_ref[...] = sm[0:1, :]


def _head_mlp(hsum, W1, b1, W2, b2):
    return pl.pallas_call(
        _head_body,
        in_specs=[
            pl.BlockSpec((1, DH), lambda: (0, 0)),
            pl.BlockSpec((DH, DH), lambda: (0, 0)),
            pl.BlockSpec((1, DH), lambda: (0, 0)),
            pl.BlockSpec((DH, NCLS), lambda: (0, 0)),
            pl.BlockSpec((1, NCLS), lambda: (0, 0)),
        ],
        out_specs=pl.BlockSpec((1, NCLS), lambda: (0, 0)),
        out_shape=jax.ShapeDtypeStruct((1, NCLS), f32),
    )(hsum, W1, b1.reshape(1, DH), W2, b2.reshape(1, NCLS))


_MESH = plsc.VectorSubcoreMesh(core_axis_name="c", subcore_axis_name="s")


@functools.partial(
    pl.kernel,
    mesh=_MESH,
    out_type=[jax.ShapeDtypeStruct((E, F), f32)] * 2,
    scratch_types=[
        pltpu.VMEM((K,), jnp.int32),
        pltpu.VMEM((K, F), f32),
        pltpu.SemaphoreType.DMA,
    ],
)
def _gather(fs_hbm, fd_hbm, src_hbm, dst_hbm, gs_hbm, gd_hbm,
            idx_v, rows_v, sem):
    wid = lax.axis_index("s") * 2 + lax.axis_index("c")
    base = wid * EPW

    @pl.loop(0, NCH)
    def _(i):
        off = base + i * K
        pltpu.sync_copy(src_hbm.at[pl.ds(off, K)], idx_v)
        pltpu.async_copy(fs_hbm.at[idx_v], rows_v, sem).wait()
        pltpu.sync_copy(rows_v, gs_hbm.at[pl.ds(off, K)])
        pltpu.sync_copy(dst_hbm.at[pl.ds(off, K)], idx_v)
        pltpu.async_copy(fd_hbm.at[idx_v], rows_v, sem).wait()
        pltpu.sync_copy(rows_v, gd_hbm.at[pl.ds(off, K)])


@functools.partial(
    pl.kernel,
    mesh=_MESH,
    compiler_params=pltpu.CompilerParams(use_tc_tiling_on_sc=False),
    out_type=[
        jax.ShapeDtypeStruct((H, 2, NPAD, DH), f32),
        jax.ShapeDtypeStruct((2, NPAD, 16), f32),
    ],
    scratch_types=[
        pltpu.VMEM((K,), jnp.int32),
        pltpu.VMEM((K, DH), f32),
        pltpu.VMEM((K, 16), f32),
        pltpu.VMEM_SHARED((NPAD, DH), f32),
        pltpu.VMEM_SHARED((NPAD, 16), f32),
    ],
)
def _aggregate(msg_hbm, ex_hbm, dst_hbm, zacc_hbm, zden_hbm,
               acc_hbm, den_hbm, idx_v, rows_v, exr_v, acc_sh, den_sh):
    cid = lax.axis_index("c")
    sid = lax.axis_index("s")
    wid = sid * 2 + cid
    base = wid * EPW
    zone = sid * NPS
    NJ = NPS // K

    pltpu.sync_copy(zden_hbm, exr_v)
    for j in range(NJ):
        pltpu.sync_copy(exr_v, den_sh.at[pl.ds(zone + j * K, K)])
    plsc.subcore_barrier()

    @pl.loop(0, NCH)
    def _(i):
        pltpu.sync_copy(dst_hbm.at[pl.ds(base + i * K, K)], idx_v)
        pltpu.sync_copy(ex_hbm.at[pl.ds(base + i * K, K)], exr_v)
        pltpu.sync_copy(exr_v, den_sh.at[idx_v], add=True)

    plsc.subcore_barrier()
    for j in range(NJ):
        pltpu.sync_copy(den_sh.at[pl.ds(zone + j * K, K)], exr_v)
        pltpu.sync_copy(exr_v, den_hbm.at[cid, pl.ds(zone + j * K, K)])

    for h in range(H):
        plsc.subcore_barrier()
        pltpu.sync_copy(zacc_hbm, rows_v)
        for j in range(NJ):
            pltpu.sync_copy(rows_v, acc_sh.at[pl.ds(zone + j * K, K)])
        plsc.subcore_barrier()

        @pl.loop(0, NCH)
        def _(i):
            pltpu.sync_copy(dst_hbm.at[pl.ds(base + i * K, K)], idx_v)
            pltpu.sync_copy(msg_hbm.at[h, pl.ds(base + i * K, K)], rows_v)
            pltpu.sync_copy(rows_v, acc_sh.at[idx_v], add=True)

        plsc.subcore_barrier()
        for j in range(NJ):
            pltpu.sync_copy(acc_sh.at[pl.ds(zone + j * K, K)], rows_v)
            pltpu.sync_copy(rows_v, acc_hbm.at[h, cid, pl.ds(zone + j * K, K)])


def _block_diag_a(a):
    rows = jnp.arange(F)
    A = jnp.zeros((F, 16), f32).at[rows, rows // DH].set(a.reshape(F))
    return A


def _gat_layer(fs, fd, src, dst, a16, zacc, zden):
    gs, gd = _gather(fs, fd, src, dst)
    msg, ex = _edge_pass(gs, gd, a16)
    return _aggregate(msg, ex, dst, zacc, zden)


def kernel(g_feats, edge_index, W_in, b_in, Ws1, bs1, Wd1, bd1, a1,
           Ws2, bs2, Wd2, bd2, a2, Wh1, bh1, Wh2, bh2):
    src = edge_index[0]
    dst = edge_index[1]
    zacc = jnp.zeros((K, DH), f32)
    zden = jnp.zeros((K, 16), f32)

    fs1, fd1 = _mm_in(g_feats, W_in, b_in, Ws1, bs1, Wd1, bd1)
    acc1, den1 = _gat_layer(fs1, fd1, src, dst, _block_diag_a(a1), zacc, zden)
    fs2, fd2 = _node_mid(acc1, den1, Ws2, bs2, Wd2, bd2)
    acc2, den2 = _gat_layer(fs2, fd2, src, dst, _block_diag_a(a2), zacc, zden)
    hsum = _reduce_nodes(acc2, den2)
    return _head_mlp(hsum, Wh1, bh1, Wh2, bh2)


# speedup vs baseline: 2.9073x; 1.3283x over previous
import functools

import jax
import jax.numpy as jnp
from jax import lax
from jax.experimental import pallas as pl
from jax.experimental.pallas import tpu as pltpu
from jax.experimental.pallas import tpu_sc as plsc

N = 10000
E = 320000
DIN = 128
DH = 64
H = 8
NCLS = 10
F = H * DH

NW = 32
EPW = E // NW
K = 80
NCH = EPW // K
NPS = 640
NPAD = 16 * NPS

f32 = jnp.float32
BN = 1000
BE = 2000


def _mm_in_body(x_ref, win_ref, bin_ref, ws_ref, bs_ref, wd_ref, bd_ref,
                fs_ref, fd_ref):
    h0 = jnp.dot(x_ref[...], win_ref[...], preferred_element_type=f32)
    h0 = h0 + bin_ref[...]
    fs_ref[...] = jnp.dot(h0, ws_ref[...], preferred_element_type=f32) + bs_ref[...]
    fd_ref[...] = jnp.dot(h0, wd_ref[...], preferred_element_type=f32) + bd_ref[...]


def _mm_in(x, Win, bin_, Ws, bs, Wd, bd):
    return pl.pallas_call(
        _mm_in_body,
        grid=(N // BN,),
        in_specs=[
            pl.BlockSpec((BN, DIN), lambda i: (i, 0)),
            pl.BlockSpec((DIN, DH), lambda i: (0, 0)),
            pl.BlockSpec((1, DH), lambda i: (0, 0)),
            pl.BlockSpec((DH, F), lambda i: (0, 0)),
            pl.BlockSpec((1, F), lambda i: (0, 0)),
            pl.BlockSpec((DH, F), lambda i: (0, 0)),
            pl.BlockSpec((1, F), lambda i: (0, 0)),
        ],
        out_specs=[pl.BlockSpec((BN, F), lambda i: (i, 0))] * 2,
        out_shape=[jax.ShapeDtypeStruct((N, F), f32)] * 2,
    )(x, Win, bin_.reshape(1, DH), Ws, bs.reshape(1, F), Wd, bd.reshape(1, F))


def _edge_body(gs_ref, gd_ref, a_ref, msg_ref, ex_ref):
    x = gs_ref[...] + gd_ref[...]
    t = jnp.maximum(x, 0.2 * x)
    logits = jnp.dot(t, a_ref[...], preferred_element_type=f32)
    col = lax.broadcasted_iota(jnp.int32, (1, 16), 1)
    ex = jnp.where(col < H, jnp.exp(logits), 0.0)
    ex_ref[...] = ex
    for h in range(H):
        msg_ref[h] = gs_ref[:, h * DH:(h + 1) * DH] * ex[:, h:h + 1]


def _edge_pass(gs, gd, a16):
    return pl.pallas_call(
        _edge_body,
        grid=(E // BE,),
        in_specs=[
            pl.BlockSpec((BE, F), lambda i: (i, 0)),
            pl.BlockSpec((BE, F), lambda i: (i, 0)),
            pl.BlockSpec((F, 16), lambda i: (0, 0)),
        ],
        out_specs=[
            pl.BlockSpec((H, BE, DH), lambda i: (0, i, 0)),
            pl.BlockSpec((BE, 16), lambda i: (i, 0)),
        ],
        out_shape=[
            jax.ShapeDtypeStruct((H, E, DH), f32),
            jax.ShapeDtypeStruct((E, 16), f32),
        ],
    )(gs, gd, a16)


def _heads_merge(acc_ref, den_ref):
    d = den_ref[0] + den_ref[1]
    s = jnp.zeros((acc_ref.shape[2], DH), f32)
    for h in range(H):
        num = acc_ref[h, 0] + acc_ref[h, 1]
        dh = d[:, h:h + 1]
        s = s + jnp.where(dh > 0, num / dh, 0.0)
    return s * (1.0 / H)


def _node_mid_body(acc_ref, den_ref, ws_ref, bs_ref, wd_ref, bd_ref,
                   fs_ref, fd_ref):
    h1 = jnp.maximum(_heads_merge(acc_ref, den_ref), 0.0)
    fs_ref[...] = jnp.dot(h1, ws_ref[...], preferred_element_type=f32) + bs_ref[...]
    fd_ref[...] = jnp.dot(h1, wd_ref[...], preferred_element_type=f32) + bd_ref[...]


def _node_mid(acc, den, Ws, bs, Wd, bd):
    return pl.pallas_call(
        _node_mid_body,
        grid=(N // BN,),
        in_specs=[
            pl.BlockSpec((H, 2, BN, DH), lambda i: (0, 0, i, 0)),
            pl.BlockSpec((2, BN, 16), lambda i: (0, i, 0)),
            pl.BlockSpec((DH, F), lambda i: (0, 0)),
            pl.BlockSpec((1, F), lambda i: (0, 0)),
            pl.BlockSpec((DH, F), lambda i: (0, 0)),
            pl.BlockSpec((1, F), lambda i: (0, 0)),
        ],
        out_specs=[pl.BlockSpec((BN, F), lambda i: (i, 0))] * 2,
        out_shape=[jax.ShapeDtypeStruct((N, F), f32)] * 2,
    )(acc, den, Ws, bs.reshape(1, F), Wd, bd.reshape(1, F))


def _reduce_body(acc_ref, den_ref, o_ref):
    h2 = _heads_merge(acc_ref, den_ref)
    part = jnp.sum(h2, axis=0, keepdims=True)

    @pl.when(pl.program_id(0) == 0)
    def _():
        o_ref[...] = jnp.zeros_like(o_ref)

    o_ref[...] += part


def _reduce_nodes(acc, den):
    return pl.pallas_call(
        _reduce_body,
        grid=(N // BN,),
        in_specs=[
            pl.BlockSpec((H, 2, BN, DH), lambda i: (0, 0, i, 0)),
            pl.BlockSpec((2, BN, 16), lambda i: (0, i, 0)),
        ],
        out_specs=pl.BlockSpec((1, DH), lambda i: (0, 0)),
        out_shape=jax.ShapeDtypeStruct((1, DH), f32),
    )(acc, den)


def _head_body(hs_ref, w1_ref, b1_ref, w2_ref, b2_ref, o_ref):
    g = jnp.broadcast_to(hs_ref[...] * (1.0 / N), (8, DH))
    a = jnp.maximum(jnp.dot(g, w1_ref[...], preferred_element_type=f32)
                    + b1_ref[...], 0.0)
    z = jnp.dot(a, w2_ref[...], preferred_element_type=f32) + b2_ref[...]
    z = z - jnp.max(z, axis=-1, keepdims=True)
    ez = jnp.exp(z)
    sm = ez / jnp.sum(ez, axis=-1, keepdims=True)
    o_ref[...] = sm[0:1, :]


def _head_mlp(hsum, W1, b1, W2, b2):
    return pl.pallas_call(
        _head_body,
        in_specs=[
            pl.BlockSpec((1, DH), lambda: (0, 0)),
            pl.BlockSpec((DH, DH), lambda: (0, 0)),
            pl.BlockSpec((1, DH), lambda: (0, 0)),
            pl.BlockSpec((DH, NCLS), lambda: (0, 0)),
            pl.BlockSpec((1, NCLS), lambda: (0, 0)),
        ],
        out_specs=pl.BlockSpec((1, NCLS), lambda: (0, 0)),
        out_shape=jax.ShapeDtypeStruct((1, NCLS), f32),
    )(hsum, W1, b1.reshape(1, DH), W2, b2.reshape(1, NCLS))


_MESH = plsc.VectorSubcoreMesh(core_axis_name="c", subcore_axis_name="s")


@functools.partial(
    pl.kernel,
    mesh=_MESH,
    out_type=[jax.ShapeDtypeStruct((E, F), f32)] * 2,
    scratch_types=[
        pltpu.VMEM((K,), jnp.int32),
        pltpu.VMEM((K,), jnp.int32),
        pltpu.VMEM((K, F), f32),
        pltpu.VMEM((K, F), f32),
        pltpu.SemaphoreType.DMA,
        pltpu.SemaphoreType.DMA,
    ],
)
def _gather(fs_hbm, fd_hbm, src_hbm, dst_hbm, gs_hbm, gd_hbm,
            idx_a, idx_b, rows_a, rows_b, sem_a, sem_b):
    wid = lax.axis_index("s") * 2 + lax.axis_index("c")
    base = wid * EPW

    for table_hbm, eidx_hbm, out_hbm in ((fs_hbm, src_hbm, gs_hbm),
                                         (fd_hbm, dst_hbm, gd_hbm)):
        def start(i, idx_v, rows_v, sem):
            pltpu.sync_copy(eidx_hbm.at[pl.ds(base + i * K, K)], idx_v)
            pltpu.async_copy(table_hbm.at[idx_v], rows_v, sem)

        def finish(i, idx_v, rows_v, sem):
            pltpu.make_async_copy(table_hbm.at[idx_v], rows_v, sem).wait()
            pltpu.sync_copy(rows_v, out_hbm.at[pl.ds(base + i * K, K)])

        start(0, idx_a, rows_a, sem_a)

        @pl.loop(0, NCH // 2)
        def _(p):
            i0 = 2 * p
            start(i0 + 1, idx_b, rows_b, sem_b)
            finish(i0, idx_a, rows_a, sem_a)
            start(i0 + 2, idx_a, rows_a, sem_a)
            finish(i0 + 1, idx_b, rows_b, sem_b)

        finish(NCH - 1, idx_a, rows_a, sem_a)


@functools.partial(
    pl.kernel,
    mesh=_MESH,
    compiler_params=pltpu.CompilerParams(use_tc_tiling_on_sc=False),
    out_type=[
        jax.ShapeDtypeStruct((H, 2, NPAD, DH), f32),
        jax.ShapeDtypeStruct((2, NPAD, 16), f32),
    ],
    scratch_types=[
        pltpu.VMEM((K,), jnp.int32),
        pltpu.VMEM((K,), jnp.int32),
        pltpu.VMEM((K, DH), f32),
        pltpu.VMEM((K, DH), f32),
        pltpu.VMEM((K, 16), f32),
        pltpu.VMEM_SHARED((NPAD, DH), f32),
        pltpu.VMEM_SHARED((NPAD, 16), f32),
        pltpu.SemaphoreType.DMA,
        pltpu.SemaphoreType.DMA,
    ],
)
def _aggregate(msg_hbm, ex_hbm, dst_hbm, zacc_hbm, zden_hbm,
               acc_hbm, den_hbm, idx_a, idx_b, rows_a, rows_b, exr_v,
               acc_sh, den_sh, sem_a, sem_b):
    cid = lax.axis_index("c")
    sid = lax.axis_index("s")
    wid = sid * 2 + cid
    base = wid * EPW
    zone = sid * NPS
    NJ = NPS // K

    pltpu.sync_copy(zden_hbm, exr_v)
    for j in range(NJ):
        pltpu.sync_copy(exr_v, den_sh.at[pl.ds(zone + j * K, K)])
    plsc.subcore_barrier()

    @pl.loop(0, NCH)
    def _(i):
        pltpu.sync_copy(dst_hbm.at[pl.ds(base + i * K, K)], idx_a)
        pltpu.sync_copy(ex_hbm.at[pl.ds(base + i * K, K)], exr_v)
        pltpu.sync_copy(exr_v, den_sh.at[idx_a], add=True)

    plsc.subcore_barrier()
    for j in range(NJ):
        pltpu.sync_copy(den_sh.at[pl.ds(zone + j * K, K)], exr_v)
        pltpu.sync_copy(exr_v, den_hbm.at[cid, pl.ds(zone + j * K, K)])

    for h in range(H):
        def issue(i, idx_v, rows_v, sem):
            pltpu.async_copy(dst_hbm.at[pl.ds(base + i * K, K)], idx_v, sem)
            pltpu.async_copy(msg_hbm.at[h, pl.ds(base + i * K, K)], rows_v, sem)

        def drain_add(i, idx_v, rows_v, sem):
            pltpu.make_async_copy(dst_hbm.at[pl.ds(base + i * K, K)],
                                  idx_v, sem).wait()
            pltpu.make_async_copy(msg_hbm.at[h, pl.ds(base + i * K, K)],
                                  rows_v, sem).wait()
            pltpu.sync_copy(rows_v, acc_sh.at[idx_v], add=True)

        plsc.subcore_barrier()
        pltpu.sync_copy(zacc_hbm, rows_a)
        for j in range(NJ):
            pltpu.sync_copy(rows_a, acc_sh.at[pl.ds(zone + j * K, K)])
        plsc.subcore_barrier()

        issue(0, idx_a, rows_a, sem_a)

        @pl.loop(0, NCH // 2)
        def _(p):
            i0 = 2 * p
            issue(i0 + 1, idx_b, rows_b, sem_b)
            drain_add(i0, idx_a, rows_a, sem_a)
            issue(i0 + 2, idx_a, rows_a, sem_a)
            drain_add(i0 + 1, idx_b, rows_b, sem_b)

        drain_add(NCH - 1, idx_a, rows_a, sem_a)

        plsc.subcore_barrier()
        for j in range(NJ):
            pltpu.sync_copy(acc_sh.at[pl.ds(zone + j * K, K)], rows_a)
            pltpu.sync_copy(rows_a, acc_hbm.at[h, cid, pl.ds(zone + j * K, K)])


def _block_diag_a(a):
    rows = jnp.arange(F)
    A = jnp.zeros((F, 16), f32).at[rows, rows // DH].set(a.reshape(F))
    return A


def _gat_layer(fs, fd, src, dst, a16, zacc, zden):
    gs, gd = _gather(fs, fd, src, dst)
    msg, ex = _edge_pass(gs, gd, a16)
    return _aggregate(msg, ex, dst, zacc, zden)


def kernel(g_feats, edge_index, W_in, b_in, Ws1, bs1, Wd1, bd1, a1,
           Ws2, bs2, Wd2, bd2, a2, Wh1, bh1, Wh2, bh2):
    src = edge_index[0]
    dst = edge_index[1]
    zacc = jnp.zeros((K, DH), f32)
    zden = jnp.zeros((K, 16), f32)

    fs1, fd1 = _mm_in(g_feats, W_in, b_in, Ws1, bs1, Wd1, bd1)
    acc1, den1 = _gat_layer(fs1, fd1, src, dst, _block_diag_a(a1), zacc, zden)
    fs2, fd2 = _node_mid(acc1, den1, Ws2, bs2, Wd2, bd2)
    acc2, den2 = _gat_layer(fs2, fd2, src, dst, _block_diag_a(a2), zacc, zden)
    hsum = _reduce_nodes(acc2, den2)
    return _head_mlp(hsum, Wh1, bh1, Wh2, bh2)


# speedup vs baseline: 3.1742x; 1.0918x over previous
import functools

import jax
import jax.numpy as jnp
from jax import lax
from jax.experimental import pallas as pl
from jax.experimental.pallas import tpu as pltpu
from jax.experimental.pallas import tpu_sc as plsc

N = 10000
E = 320000
DIN = 128
DH = 64
H = 8
NCLS = 10
F = H * DH

NW = 32
EPW = E // NW
K = 80
NCH = EPW // K
NPS = 640
NPAD = 16 * NPS

f32 = jnp.float32
BN = 1000
BE = 2000


def _mm_in_body(x_ref, win_ref, bin_ref, ws_ref, bs_ref, wd_ref, bd_ref,
                fs_ref, fd_ref):
    h0 = jnp.dot(x_ref[...], win_ref[...], preferred_element_type=f32)
    h0 = h0 + bin_ref[...]
    fs_ref[...] = jnp.dot(h0, ws_ref[...], preferred_element_type=f32) + bs_ref[...]
    fd_ref[...] = jnp.dot(h0, wd_ref[...], preferred_element_type=f32) + bd_ref[...]


def _mm_in(x, Win, bin_, Ws, bs, Wd, bd):
    return pl.pallas_call(
        _mm_in_body,
        grid=(N // BN,),
        in_specs=[
            pl.BlockSpec((BN, DIN), lambda i: (i, 0)),
            pl.BlockSpec((DIN, DH), lambda i: (0, 0)),
            pl.BlockSpec((1, DH), lambda i: (0, 0)),
            pl.BlockSpec((DH, F), lambda i: (0, 0)),
            pl.BlockSpec((1, F), lambda i: (0, 0)),
            pl.BlockSpec((DH, F), lambda i: (0, 0)),
            pl.BlockSpec((1, F), lambda i: (0, 0)),
        ],
        out_specs=[pl.BlockSpec((BN, F), lambda i: (i, 0))] * 2,
        out_shape=[jax.ShapeDtypeStruct((N, F), f32)] * 2,
    )(x, Win, bin_.reshape(1, DH), Ws, bs.reshape(1, F), Wd, bd.reshape(1, F))


def _edge_body(gs_ref, gd_ref, a_ref, msg_ref, ex_ref):
    x = gs_ref[...] + gd_ref[...]
    t = jnp.maximum(x, 0.2 * x)
    logits = jnp.dot(t, a_ref[...], preferred_element_type=f32)
    col = lax.broadcasted_iota(jnp.int32, (1, 16), 1)
    ex = jnp.where(col < H, jnp.exp(logits), 0.0)
    ex_ref[...] = ex
    for h in range(H):
        msg_ref[h] = gs_ref[:, h * DH:(h + 1) * DH] * ex[:, h:h + 1]


def _edge_pass(gs, gd, a16, ecount):
    return pl.pallas_call(
        _edge_body,
        grid=(ecount // BE,),
        in_specs=[
            pl.BlockSpec((BE, F), lambda i: (i, 0)),
            pl.BlockSpec((BE, F), lambda i: (i, 0)),
            pl.BlockSpec((F, 16), lambda i: (0, 0)),
        ],
        out_specs=[
            pl.BlockSpec((H, BE, DH), lambda i: (0, i, 0)),
            pl.BlockSpec((BE, 16), lambda i: (i, 0)),
        ],
        out_shape=[
            jax.ShapeDtypeStruct((H, ecount, DH), f32),
            jax.ShapeDtypeStruct((ecount, 16), f32),
        ],
    )(gs, gd, a16)


def _heads_merge(acca_ref, accb_ref, dena_ref, denb_ref):
    d = dena_ref[0] + dena_ref[1] + denb_ref[0] + denb_ref[1]
    s = jnp.zeros((acca_ref.shape[2], DH), f32)
    for h in range(H):
        num = (acca_ref[h, 0] + acca_ref[h, 1]
               + accb_ref[h, 0] + accb_ref[h, 1])
        dh = d[:, h:h + 1]
        s = s + jnp.where(dh > 0, num / dh, 0.0)
    return s * (1.0 / H)


def _node_mid_body(acca_ref, accb_ref, dena_ref, denb_ref,
                   ws_ref, bs_ref, wd_ref, bd_ref, fs_ref, fd_ref):
    h1 = jnp.maximum(_heads_merge(acca_ref, accb_ref, dena_ref, denb_ref), 0.0)
    fs_ref[...] = jnp.dot(h1, ws_ref[...], preferred_element_type=f32) + bs_ref[...]
    fd_ref[...] = jnp.dot(h1, wd_ref[...], preferred_element_type=f32) + bd_ref[...]


def _node_mid(acca, accb, dena, denb, Ws, bs, Wd, bd):
    return pl.pallas_call(
        _node_mid_body,
        grid=(N // BN,),
        in_specs=[
            pl.BlockSpec((H, 2, BN, DH), lambda i: (0, 0, i, 0)),
            pl.BlockSpec((H, 2, BN, DH), lambda i: (0, 0, i, 0)),
            pl.BlockSpec((2, BN, 16), lambda i: (0, i, 0)),
            pl.BlockSpec((2, BN, 16), lambda i: (0, i, 0)),
            pl.BlockSpec((DH, F), lambda i: (0, 0)),
            pl.BlockSpec((1, F), lambda i: (0, 0)),
            pl.BlockSpec((DH, F), lambda i: (0, 0)),
            pl.BlockSpec((1, F), lambda i: (0, 0)),
        ],
        out_specs=[pl.BlockSpec((BN, F), lambda i: (i, 0))] * 2,
        out_shape=[jax.ShapeDtypeStruct((N, F), f32)] * 2,
    )(acca, accb, dena, denb, Ws, bs.reshape(1, F), Wd, bd.reshape(1, F))


def _reduce_body(acca_ref, accb_ref, dena_ref, denb_ref, o_ref):
    h2 = _heads_merge(acca_ref, accb_ref, dena_ref, denb_ref)
    part = jnp.sum(h2, axis=0, keepdims=True)

    @pl.when(pl.program_id(0) == 0)
    def _():
        o_ref[...] = jnp.zeros_like(o_ref)

    o_ref[...] += part


def _reduce_nodes(acca, accb, dena, denb):
    return pl.pallas_call(
        _reduce_body,
        grid=(N // BN,),
        in_specs=[
            pl.BlockSpec((H, 2, BN, DH), lambda i: (0, 0, i, 0)),
            pl.BlockSpec((H, 2, BN, DH), lambda i: (0, 0, i, 0)),
            pl.BlockSpec((2, BN, 16), lambda i: (0, i, 0)),
            pl.BlockSpec((2, BN, 16), lambda i: (0, i, 0)),
        ],
        out_specs=pl.BlockSpec((1, DH), lambda i: (0, 0)),
        out_shape=jax.ShapeDtypeStruct((1, DH), f32),
    )(acca, accb, dena, denb)


def _head_body(hs_ref, w1_ref, b1_ref, w2_ref, b2_ref, o_ref):
    g = jnp.broadcast_to(hs_ref[...] * (1.0 / N), (8, DH))
    a = jnp.maximum(jnp.dot(g, w1_ref[...], preferred_element_type=f32)
                    + b1_ref[...], 0.0)
    z = jnp.dot(a, w2_ref[...], preferred_element_type=f32) + b2_ref[...]
    z = z - jnp.max(z, axis=-1, keepdims=True)
    ez = jnp.exp(z)
    sm = ez / jnp.sum(ez, axis=-1, keepdims=True)
    o_ref[...] = sm[0:1, :]


def _head_mlp(hsum, W1, b1, W2, b2):
    return pl.pallas_call(
        _head_body,
        in_specs=[
            pl.BlockSpec((1, DH), lambda: (0, 0)),
            pl.BlockSpec((DH, DH), lambda: (0, 0)),
            pl.BlockSpec((1, DH), lambda: (0, 0)),
            pl.BlockSpec((DH, NCLS), lambda: (0, 0)),
            pl.BlockSpec((1, NCLS), lambda: (0, 0)),
        ],
        out_specs=pl.BlockSpec((1, NCLS), lambda: (0, 0)),
        out_shape=jax.ShapeDtypeStruct((1, NCLS), f32),
    )(hsum, W1, b1.reshape(1, DH), W2, b2.reshape(1, NCLS))


_MESH = plsc.VectorSubcoreMesh(core_axis_name="c", subcore_axis_name="s")


def _pipeline_chunks(nch, start, finish):
    start(0, 0)

    @pl.loop(0, nch // 2)
    def _(p):
        i0 = 2 * p
        start(i0 + 1, 1)
        finish(i0, 0)
        if nch % 2 == 1:
            start(i0 + 2, 0)
        else:
            @pl.when(i0 + 2 < nch)
            def _():
                start(i0 + 2, 0)
        finish(i0 + 1, 1)

    if nch % 2 == 1:
        finish(nch - 1, 0)


def _make_gather(epw):
    nch = epw // K

    @functools.partial(
        pl.kernel,
        mesh=_MESH,
        out_type=[jax.ShapeDtypeStruct((NW * epw, F), f32)] * 2,
        scratch_types=[
            pltpu.VMEM((K,), jnp.int32),
            pltpu.VMEM((K,), jnp.int32),
            pltpu.VMEM((K, F), f32),
            pltpu.VMEM((K, F), f32),
            pltpu.SemaphoreType.DMA,
            pltpu.SemaphoreType.DMA,
        ],
    )
    def gather(fs_hbm, fd_hbm, src_hbm, dst_hbm, gs_hbm, gd_hbm,
               idx_a, idx_b, rows_a, rows_b, sem_a, sem_b):
        wid = lax.axis_index("s") * 2 + lax.axis_index("c")
        base = wid * epw
        idxs = (idx_a, idx_b)
        rows = (rows_a, rows_b)
        sems = (sem_a, sem_b)

        for table_hbm, eidx_hbm, out_hbm in ((fs_hbm, src_hbm, gs_hbm),
                                             (fd_hbm, dst_hbm, gd_hbm)):
            def start(i, b):
                pltpu.sync_copy(eidx_hbm.at[pl.ds(base + i * K, K)], idxs[b])
                pltpu.async_copy(table_hbm.at[idxs[b]], rows[b], sems[b])

            def finish(i, b):
                pltpu.make_async_copy(table_hbm.at[idxs[b]], rows[b],
                                      sems[b]).wait()
                pltpu.sync_copy(rows[b], out_hbm.at[pl.ds(base + i * K, K)])

            _pipeline_chunks(nch, start, finish)

    return gather


def _make_aggregate(epw):
    nch = epw // K

    @functools.partial(
        pl.kernel,
        mesh=_MESH,
        compiler_params=pltpu.CompilerParams(use_tc_tiling_on_sc=False),
        out_type=[
            jax.ShapeDtypeStruct((H, 2, NPAD, DH), f32),
            jax.ShapeDtypeStruct((2, NPAD, 16), f32),
        ],
        scratch_types=[
            pltpu.VMEM((K,), jnp.int32),
            pltpu.VMEM((K,), jnp.int32),
            pltpu.VMEM((K, DH), f32),
            pltpu.VMEM((K, DH), f32),
            pltpu.VMEM((K, 16), f32),
            pltpu.VMEM_SHARED((NPAD, DH), f32),
            pltpu.VMEM_SHARED((NPAD, 16), f32),
            pltpu.SemaphoreType.DMA,
            pltpu.SemaphoreType.DMA,
        ],
    )
    def aggregate(msg_hbm, ex_hbm, dst_hbm, zacc_hbm, zden_hbm,
                  acc_hbm, den_hbm, idx_a, idx_b, rows_a, rows_b, exr_v,
                  acc_sh, den_sh, sem_a, sem_b):
        cid = lax.axis_index("c")
        sid = lax.axis_index("s")
        wid = sid * 2 + cid
        base = wid * epw
        zone = sid * NPS
        idxs = (idx_a, idx_b)
        rows = (rows_a, rows_b)
        sems = (sem_a, sem_b)
        NJ = NPS // K

        pltpu.sync_copy(zden_hbm, exr_v)
        for j in range(NJ):
            pltpu.sync_copy(exr_v, den_sh.at[pl.ds(zone + j * K, K)])
        plsc.subcore_barrier()

        @pl.loop(0, nch)
        def _(i):
            pltpu.sync_copy(dst_hbm.at[pl.ds(base + i * K, K)], idx_a)
            pltpu.sync_copy(ex_hbm.at[pl.ds(base + i * K, K)], exr_v)
            pltpu.sync_copy(exr_v, den_sh.at[idx_a], add=True)

        plsc.subcore_barrier()
        for j in range(NJ):
            pltpu.sync_copy(den_sh.at[pl.ds(zone + j * K, K)], exr_v)
            pltpu.sync_copy(exr_v, den_hbm.at[cid, pl.ds(zone + j * K, K)])

        for h in range(H):
            def start(i, b):
                pltpu.async_copy(dst_hbm.at[pl.ds(base + i * K, K)],
                                 idxs[b], sems[b])
                pltpu.async_copy(msg_hbm.at[h, pl.ds(base + i * K, K)],
                                 rows[b], sems[b])

            def finish(i, b):
                pltpu.make_async_copy(dst_hbm.at[pl.ds(base + i * K, K)],
                                      idxs[b], sems[b]).wait()
                pltpu.make_async_copy(msg_hbm.at[h, pl.ds(base + i * K, K)],
                                      rows[b], sems[b]).wait()
                pltpu.sync_copy(rows[b], acc_sh.at[idxs[b]], add=True)

            plsc.subcore_barrier()
            pltpu.sync_copy(zacc_hbm, rows_a)
            for j in range(NJ):
                pltpu.sync_copy(rows_a, acc_sh.at[pl.ds(zone + j * K, K)])
            plsc.subcore_barrier()

            _pipeline_chunks(nch, start, finish)

            plsc.subcore_barrier()
            for j in range(NJ):
                pltpu.sync_copy(acc_sh.at[pl.ds(zone + j * K, K)], rows_a)
                pltpu.sync_copy(rows_a,
                                acc_hbm.at[h, cid, pl.ds(zone + j * K, K)])

    return aggregate


def _block_diag_a(a):
    rows = jnp.arange(F)
    A = jnp.zeros((F, 16), f32).at[rows, rows // DH].set(a.reshape(F))
    return A


EA = 128000
EB = E - EA

_gather_a = _make_gather(EA // NW)
_gather_b = _make_gather(EB // NW)
_agg_a = _make_aggregate(EA // NW)
_agg_b = _make_aggregate(EB // NW)


def _gat_layer(fs, fd, srca, dsta, srcb, dstb, a16, zacc, zden):
    gsa, gda = _gather_a(fs, fd, srca, dsta)
    gsb, gdb = _gather_b(fs, fd, srcb, dstb)
    msga, exa = _edge_pass(gsa, gda, a16, EA)
    acca, dena = _agg_a(msga, exa, dsta, zacc, zden)
    msgb, exb = _edge_pass(gsb, gdb, a16, EB)
    accb, denb = _agg_b(msgb, exb, dstb, zacc, zden)
    return acca, accb, dena, denb


def kernel(g_feats, edge_index, W_in, b_in, Ws1, bs1, Wd1, bd1, a1,
           Ws2, bs2, Wd2, bd2, a2, Wh1, bh1, Wh2, bh2):
    src = edge_index[0]
    dst = edge_index[1]
    srca, srcb = src[:EA], src[EA:]
    dsta, dstb = dst[:EA], dst[EA:]
    zacc = jnp.zeros((K, DH), f32)
    zden = jnp.zeros((K, 16), f32)

    fs1, fd1 = _mm_in(g_feats, W_in, b_in, Ws1, bs1, Wd1, bd1)
    aa1, ab1, da1, db1 = _gat_layer(fs1, fd1, srca, dsta, srcb, dstb,
                                    _block_diag_a(a1), zacc, zden)
    fs2, fd2 = _node_mid(aa1, ab1, da1, db1, Ws2, bs2, Wd2, bd2)
    aa2, ab2, da2, db2 = _gat_layer(fs2, fd2, srca, dsta, srcb, dstb,
                                    _block_diag_a(a2), zacc, zden)
    hsum = _reduce_nodes(aa2, ab2, da2, db2)
    return _head_mlp(hsum, Wh1, bh1, Wh2, bh2)


# speedup vs baseline: 4.8482x; 1.5274x over previous
import functools

import jax
import jax.numpy as jnp
from jax import lax
from jax.experimental import pallas as pl
from jax.experimental.pallas import tpu as pltpu
from jax.experimental.pallas import tpu_sc as plsc

N = 10000
E = 320000
DIN = 128
DH = 64
H = 8
NCLS = 10
F = H * DH

NW = 32
EPW = E // NW
K = 80
NCH = EPW // K
NPS = 640
NPAD = 16 * NPS

f32 = jnp.float32
BN = 1000
BE = 2000


def _mm_in_body(x_ref, win_ref, bin_ref, ws_ref, bs_ref, wd_ref, bd_ref,
                fs_ref, fd_ref):
    h0 = jnp.dot(x_ref[...], win_ref[...], preferred_element_type=f32)
    h0 = h0 + bin_ref[...]
    fs_ref[...] = jnp.dot(h0, ws_ref[...], preferred_element_type=f32) + bs_ref[...]
    fd_ref[...] = jnp.dot(h0, wd_ref[...], preferred_element_type=f32) + bd_ref[...]


def _mm_in(x, Win, bin_, Ws, bs, Wd, bd):
    return pl.pallas_call(
        _mm_in_body,
        grid=(N // BN,),
        in_specs=[
            pl.BlockSpec((BN, DIN), lambda i: (i, 0)),
            pl.BlockSpec((DIN, DH), lambda i: (0, 0)),
            pl.BlockSpec((1, DH), lambda i: (0, 0)),
            pl.BlockSpec((DH, F), lambda i: (0, 0)),
            pl.BlockSpec((1, F), lambda i: (0, 0)),
            pl.BlockSpec((DH, F), lambda i: (0, 0)),
            pl.BlockSpec((1, F), lambda i: (0, 0)),
        ],
        out_specs=[pl.BlockSpec((BN, F), lambda i: (i, 0))] * 2,
        out_shape=[jax.ShapeDtypeStruct((N, F), f32)] * 2,
    )(x, Win, bin_.reshape(1, DH), Ws, bs.reshape(1, F), Wd, bd.reshape(1, F))


def _edge_body(gs_ref, gd_ref, a_ref, msg_ref, ex_ref):
    x = gs_ref[...] + gd_ref[...]
    t = jnp.maximum(x, 0.2 * x)
    logits = jnp.dot(t, a_ref[...], preferred_element_type=f32)
    col = lax.broadcasted_iota(jnp.int32, (1, 16), 1)
    ex = jnp.where(col < H, jnp.exp(logits), 0.0)
    ex_ref[...] = ex
    for k in range(H // 2):
        lo = gs_ref[:, (2 * k) * DH:(2 * k + 1) * DH] * ex[:, 2 * k:2 * k + 1]
        hi = gs_ref[:, (2 * k + 1) * DH:(2 * k + 2) * DH] * ex[:, 2 * k + 1:2 * k + 2]
        msg_ref[k] = jnp.concatenate([lo, hi], axis=1)


def _edge_pass(gs, gd, a16, ecount):
    return pl.pallas_call(
        _edge_body,
        grid=(ecount // BE,),
        in_specs=[
            pl.BlockSpec((BE, F), lambda i: (i, 0)),
            pl.BlockSpec((BE, F), lambda i: (i, 0)),
            pl.BlockSpec((F, 16), lambda i: (0, 0)),
        ],
        out_specs=[
            pl.BlockSpec((H // 2, BE, 2 * DH), lambda i: (0, i, 0)),
            pl.BlockSpec((BE, 16), lambda i: (i, 0)),
        ],
        out_shape=[
            jax.ShapeDtypeStruct((H // 2, ecount, 2 * DH), f32),
            jax.ShapeDtypeStruct((ecount, 16), f32),
        ],
    )(gs, gd, a16)


def _heads_merge(acca_ref, accb_ref, dena_ref, denb_ref):
    d = dena_ref[0] + dena_ref[1] + denb_ref[0] + denb_ref[1]
    s = jnp.zeros((acca_ref.shape[2], DH), f32)
    for h in range(H):
        hp, half = h // 2, (h % 2) * DH
        num = (acca_ref[hp, 0, :, half:half + DH]
               + acca_ref[hp, 1, :, half:half + DH]
               + accb_ref[hp, 0, :, half:half + DH]
               + accb_ref[hp, 1, :, half:half + DH])
        dh = d[:, h:h + 1]
        s = s + jnp.where(dh > 0, num / dh, 0.0)
    return s * (1.0 / H)


def _node_mid_body(acca_ref, accb_ref, dena_ref, denb_ref,
                   ws_ref, bs_ref, wd_ref, bd_ref, fs_ref, fd_ref):
    h1 = jnp.maximum(_heads_merge(acca_ref, accb_ref, dena_ref, denb_ref), 0.0)
    fs_ref[...] = jnp.dot(h1, ws_ref[...], preferred_element_type=f32) + bs_ref[...]
    fd_ref[...] = jnp.dot(h1, wd_ref[...], preferred_element_type=f32) + bd_ref[...]


def _node_mid(acca, accb, dena, denb, Ws, bs, Wd, bd):
    return pl.pallas_call(
        _node_mid_body,
        grid=(N // BN,),
        in_specs=[
            pl.BlockSpec((H // 2, 2, BN, 2 * DH), lambda i: (0, 0, i, 0)),
            pl.BlockSpec((H // 2, 2, BN, 2 * DH), lambda i: (0, 0, i, 0)),
            pl.BlockSpec((2, BN, 16), lambda i: (0, i, 0)),
            pl.BlockSpec((2, BN, 16), lambda i: (0, i, 0)),
            pl.BlockSpec((DH, F), lambda i: (0, 0)),
            pl.BlockSpec((1, F), lambda i: (0, 0)),
            pl.BlockSpec((DH, F), lambda i: (0, 0)),
            pl.BlockSpec((1, F), lambda i: (0, 0)),
        ],
        out_specs=[pl.BlockSpec((BN, F), lambda i: (i, 0))] * 2,
        out_shape=[jax.ShapeDtypeStruct((N, F), f32)] * 2,
    )(acca, accb, dena, denb, Ws, bs.reshape(1, F), Wd, bd.reshape(1, F))


def _reduce_body(acca_ref, accb_ref, dena_ref, denb_ref, o_ref):
    h2 = _heads_merge(acca_ref, accb_ref, dena_ref, denb_ref)
    part = jnp.sum(h2, axis=0, keepdims=True)

    @pl.when(pl.program_id(0) == 0)
    def _():
        o_ref[...] = jnp.zeros_like(o_ref)

    o_ref[...] += part


def _reduce_nodes(acca, accb, dena, denb):
    return pl.pallas_call(
        _reduce_body,
        grid=(N // BN,),
        in_specs=[
            pl.BlockSpec((H // 2, 2, BN, 2 * DH), lambda i: (0, 0, i, 0)),
            pl.BlockSpec((H // 2, 2, BN, 2 * DH), lambda i: (0, 0, i, 0)),
            pl.BlockSpec((2, BN, 16), lambda i: (0, i, 0)),
            pl.BlockSpec((2, BN, 16), lambda i: (0, i, 0)),
        ],
        out_specs=pl.BlockSpec((1, DH), lambda i: (0, 0)),
        out_shape=jax.ShapeDtypeStruct((1, DH), f32),
    )(acca, accb, dena, denb)


def _head_body(hs_ref, w1_ref, b1_ref, w2_ref, b2_ref, o_ref):
    g = jnp.broadcast_to(hs_ref[...] * (1.0 / N), (8, DH))
    a = jnp.maximum(jnp.dot(g, w1_ref[...], preferred_element_type=f32)
                    + b1_ref[...], 0.0)
    z = jnp.dot(a, w2_ref[...], preferred_element_type=f32) + b2_ref[...]
    z = z - jnp.max(z, axis=-1, keepdims=True)
    ez = jnp.exp(z)
    sm = ez / jnp.sum(ez, axis=-1, keepdims=True)
    o_ref[...] = sm[0:1, :]


def _head_mlp(hsum, W1, b1, W2, b2):
    return pl.pallas_call(
        _head_body,
        in_specs=[
            pl.BlockSpec((1, DH), lambda: (0, 0)),
            pl.BlockSpec((DH, DH), lambda: (0, 0)),
            pl.BlockSpec((1, DH), lambda: (0, 0)),
            pl.BlockSpec((DH, NCLS), lambda: (0, 0)),
            pl.BlockSpec((1, NCLS), lambda: (0, 0)),
        ],
        out_specs=pl.BlockSpec((1, NCLS), lambda: (0, 0)),
        out_shape=jax.ShapeDtypeStruct((1, NCLS), f32),
    )(hsum, W1, b1.reshape(1, DH), W2, b2.reshape(1, NCLS))


_MESH = plsc.VectorSubcoreMesh(core_axis_name="c", subcore_axis_name="s")


def _pipeline_chunks(nch, start, finish):
    start(0, 0)

    @pl.loop(0, nch // 2)
    def _(p):
        i0 = 2 * p
        start(i0 + 1, 1)
        finish(i0, 0)
        if nch % 2 == 1:
            start(i0 + 2, 0)
        else:
            @pl.when(i0 + 2 < nch)
            def _():
                start(i0 + 2, 0)
        finish(i0 + 1, 1)

    if nch % 2 == 1:
        finish(nch - 1, 0)


def _make_gather(epw):
    nch = epw // K

    @functools.partial(
        pl.kernel,
        mesh=_MESH,
        out_type=[jax.ShapeDtypeStruct((NW * epw, F), f32)] * 2,
        scratch_types=[
            pltpu.VMEM((K,), jnp.int32),
            pltpu.VMEM((K,), jnp.int32),
            pltpu.VMEM((K, F), f32),
            pltpu.VMEM((K, F), f32),
            pltpu.SemaphoreType.DMA,
            pltpu.SemaphoreType.DMA,
        ],
    )
    def gather(fs_hbm, fd_hbm, src_hbm, dst_hbm, gs_hbm, gd_hbm,
               idx_a, idx_b, rows_a, rows_b, sem_a, sem_b):
        wid = lax.axis_index("s") * 2 + lax.axis_index("c")
        base = wid * epw
        idxs = (idx_a, idx_b)
        rows = (rows_a, rows_b)
        sems = (sem_a, sem_b)

        for table_hbm, eidx_hbm, out_hbm in ((fs_hbm, src_hbm, gs_hbm),
                                             (fd_hbm, dst_hbm, gd_hbm)):
            def start(i, b):
                pltpu.sync_copy(eidx_hbm.at[pl.ds(base + i * K, K)], idxs[b])
                pltpu.async_copy(table_hbm.at[idxs[b]], rows[b], sems[b])

            def finish(i, b):
                pltpu.make_async_copy(table_hbm.at[idxs[b]], rows[b],
                                      sems[b]).wait()
                pltpu.sync_copy(rows[b], out_hbm.at[pl.ds(base + i * K, K)])

            _pipeline_chunks(nch, start, finish)

    return gather


def _make_aggregate(epw):
    nch = epw // K

    @functools.partial(
        pl.kernel,
        mesh=_MESH,
        compiler_params=pltpu.CompilerParams(use_tc_tiling_on_sc=False),
        out_type=[
            jax.ShapeDtypeStruct((H // 2, 2, NPAD, 2 * DH), f32),
            jax.ShapeDtypeStruct((2, NPAD, 16), f32),
        ],
        scratch_types=[
            pltpu.VMEM((K,), jnp.int32),
            pltpu.VMEM((K,), jnp.int32),
            pltpu.VMEM((K, 2 * DH), f32),
            pltpu.VMEM((K, 2 * DH), f32),
            pltpu.VMEM((K, 16), f32),
            pltpu.VMEM_SHARED((NPAD, 2 * DH), f32),
            pltpu.VMEM_SHARED((NPAD, 16), f32),
            pltpu.SemaphoreType.DMA,
            pltpu.SemaphoreType.DMA,
        ],
    )
    def aggregate(msg_hbm, ex_hbm, dst_hbm, zacc_hbm, zden_hbm,
                  acc_hbm, den_hbm, idx_a, idx_b, rows_a, rows_b, exr_v,
                  acc_sh, den_sh, sem_a, sem_b):
        cid = lax.axis_index("c")
        sid = lax.axis_index("s")
        wid = sid * 2 + cid
        base = wid * epw
        zone = sid * NPS
        idxs = (idx_a, idx_b)
        rows = (rows_a, rows_b)
        sems = (sem_a, sem_b)
        NJ = NPS // K

        pltpu.sync_copy(zden_hbm, exr_v)
        for j in range(NJ):
            pltpu.sync_copy(exr_v, den_sh.at[pl.ds(zone + j * K, K)])
        plsc.subcore_barrier()

        @pl.loop(0, nch)
        def _(i):
            pltpu.sync_copy(dst_hbm.at[pl.ds(base + i * K, K)], idx_a)
            pltpu.sync_copy(ex_hbm.at[pl.ds(base + i * K, K)], exr_v)
            pltpu.sync_copy(exr_v, den_sh.at[idx_a], add=True)

        plsc.subcore_barrier()
        for j in range(NJ):
            pltpu.sync_copy(den_sh.at[pl.ds(zone + j * K, K)], exr_v)
            pltpu.sync_copy(exr_v, den_hbm.at[cid, pl.ds(zone + j * K, K)])

        for hp in range(H // 2):
            def start(i, b):
                pltpu.async_copy(dst_hbm.at[pl.ds(base + i * K, K)],
                                 idxs[b], sems[b])
                pltpu.async_copy(msg_hbm.at[hp, pl.ds(base + i * K, K)],
                                 rows[b], sems[b])

            def finish(i, b):
                pltpu.make_async_copy(dst_hbm.at[pl.ds(base + i * K, K)],
                                      idxs[b], sems[b]).wait()
                pltpu.make_async_copy(msg_hbm.at[hp, pl.ds(base + i * K, K)],
                                      rows[b], sems[b]).wait()
                pltpu.sync_copy(rows[b], acc_sh.at[idxs[b]], add=True)

            plsc.subcore_barrier()
            pltpu.sync_copy(zacc_hbm, rows_a)
            for j in range(NJ):
                pltpu.sync_copy(rows_a, acc_sh.at[pl.ds(zone + j * K, K)])
            plsc.subcore_barrier()

            _pipeline_chunks(nch, start, finish)

            plsc.subcore_barrier()
            for j in range(NJ):
                pltpu.sync_copy(acc_sh.at[pl.ds(zone + j * K, K)], rows_a)
                pltpu.sync_copy(rows_a,
                                acc_hbm.at[hp, cid, pl.ds(zone + j * K, K)])

    return aggregate


def _block_diag_a(a):
    rows = jnp.arange(F)
    A = jnp.zeros((F, 16), f32).at[rows, rows // DH].set(a.reshape(F))
    return A


EA = 128000
EB = E - EA

_gather_a = _make_gather(EA // NW)
_gather_b = _make_gather(EB // NW)
_agg_a = _make_aggregate(EA // NW)
_agg_b = _make_aggregate(EB // NW)


def _gat_layer(fs, fd, srca, dsta, srcb, dstb, a16, zacc, zden):
    gsa, gda = _gather_a(fs, fd, srca, dsta)
    gsb, gdb = _gather_b(fs, fd, srcb, dstb)
    msga, exa = _edge_pass(gsa, gda, a16, EA)
    acca, dena = _agg_a(msga, exa, dsta, zacc, zden)
    msgb, exb = _edge_pass(gsb, gdb, a16, EB)
    accb, denb = _agg_b(msgb, exb, dstb, zacc, zden)
    return acca, accb, dena, denb


def kernel(g_feats, edge_index, W_in, b_in, Ws1, bs1, Wd1, bd1, a1,
           Ws2, bs2, Wd2, bd2, a2, Wh1, bh1, Wh2, bh2):
    src = edge_index[0]
    dst = edge_index[1]
    srca, srcb = src[:EA], src[EA:]
    dsta, dstb = dst[:EA], dst[EA:]
    zacc = jnp.zeros((K, 2 * DH), f32)
    zden = jnp.zeros((K, 16), f32)

    fs1, fd1 = _mm_in(g_feats, W_in, b_in, Ws1, bs1, Wd1, bd1)
    aa1, ab1, da1, db1 = _gat_layer(fs1, fd1, srca, dsta, srcb, dstb,
                                    _block_diag_a(a1), zacc, zden)
    fs2, fd2 = _node_mid(aa1, ab1, da1, db1, Ws2, bs2, Wd2, bd2)
    aa2, ab2, da2, db2 = _gat_layer(fs2, fd2, srca, dsta, srcb, dstb,
                                    _block_diag_a(a2), zacc, zden)
    hsum = _reduce_nodes(aa2, ab2, da2, db2)
    return _head_mlp(hsum, Wh1, bh1, Wh2, bh2)
